# Initial kernel scaffold; baseline (speedup 1.0000x reference)
import functools

import jax
import jax.numpy as jnp
from jax import lax
from jax.experimental import pallas as pl
from jax.experimental.pallas import tpu as pltpu
from jax.experimental.pallas import tpu_sc as plsc

B = 32
N_OBJ = 16
D = 32
N_NEURAL = 50000
N_ATOMS = 100000

TBLK = 2048
NJ = (N_ATOMS + TBLK - 1) // TBLK
NEURAL_HI = N_NEURAL + 2
NBLK_NEURAL = (NEURAL_HI + TBLK - 1) // TBLK
WPAD = NBLK_NEURAL * TBLK
NBINS = NJ * TBLK

N_BASE_PAD = 5120
PER_SC = N_BASE_PAD // 2
PER_SUB = PER_SC // 16
CHUNK = PER_SUB // 2
ZSPAN = NBINS // 16


def _hist_body(idx_hbm, out_hbm, idx_a, idx_b, ones_v, buf_v, hist_sh):
    cid = lax.axis_index("c")
    sid = lax.axis_index("s")

    for i in range(CHUNK // 16):
        ones_v[pl.ds(i * 16, 16)] = jnp.full((16,), 1.0, jnp.float32)

    def _zero(i, carry):
        buf_v[pl.ds(i * 16, 16)] = jnp.zeros((16,), jnp.float32)
        return carry

    lax.fori_loop(0, ZSPAN // 16, _zero, 0)

    zbase = sid * ZSPAN
    pltpu.sync_copy(buf_v, hist_sh.at[pl.ds(zbase, ZSPAN)])
    plsc.subcore_barrier()

    base = cid * PER_SC + sid * PER_SUB
    pltpu.sync_copy(idx_hbm.at[pl.ds(base, CHUNK)], idx_a)
    pltpu.sync_copy(idx_hbm.at[pl.ds(base + CHUNK, CHUNK)], idx_b)
    pltpu.sync_copy(ones_v, hist_sh.at[idx_a], add=True)
    pltpu.sync_copy(ones_v, hist_sh.at[idx_b], add=True)
    plsc.subcore_barrier()

    pltpu.sync_copy(hist_sh.at[pl.ds(zbase, ZSPAN)],
                    out_hbm.at[cid, pl.ds(zbase, ZSPAN)])


@functools.partial(
    pl.kernel,
    out_type=jax.ShapeDtypeStruct((2, NBINS), jnp.float32),
    mesh=plsc.VectorSubcoreMesh(core_axis_name="c", subcore_axis_name="s"),
    scratch_types=[
        pltpu.VMEM((CHUNK,), jnp.int32),
        pltpu.VMEM((CHUNK,), jnp.int32),
        pltpu.VMEM((CHUNK,), jnp.float32),
        pltpu.VMEM((ZSPAN,), jnp.float32),
        pltpu.VMEM_SHARED((NBINS,), jnp.float32),
    ],
)
def _hist_kernel(idx_hbm, out_hbm, idx_a, idx_b, ones_v, buf_v, hist_sh):
    _hist_body(idx_hbm, out_hbm, idx_a, idx_b, ones_v, buf_v, hist_sh)


def _tc_body(zf_ref, w_ref, obj_ref, cnt_ref, out_ref):
    j = pl.program_id(0)
    csum = cnt_ref[0:1, :] + cnt_ref[1:2, :]
    col = j * TBLK + lax.broadcasted_iota(jnp.int32, (1, TBLK), 1)

    @pl.when(j < NBLK_NEURAL)
    def _():
        wblk = w_ref[...]
        obj = obj_ref[0]
        wtile = jnp.concatenate([wblk] * N_OBJ, axis=0)
        oid = lax.shift_right_logical(
            lax.broadcasted_iota(jnp.int32, (N_OBJ * D, TBLK), 0), 5)
        u = jnp.where(oid == obj, wtile, 0.0)
        acc = jnp.dot(zf_ref[...], u, preferred_element_type=jnp.float32)
        vals = jax.nn.sigmoid(acc)
        neural = (col >= 2) & (col < NEURAL_HI)
        out = jnp.where(neural, vals, 0.0) + csum
        out_ref[...] = jnp.where(col == 1, 1.0, out)

    @pl.when(j >= NBLK_NEURAL)
    def _():
        out_ref[...] = jnp.broadcast_to(csum, (B, TBLK))


def kernel(Z, atom_idx, obj_idx, w, base_idx):
    del atom_idx
    zflat = Z.reshape(B, N_OBJ * D)
    wtp = jnp.pad(w.astype(jnp.float32).T, ((0, 0), (2, WPAD - NEURAL_HI)))
    obj3 = jnp.pad(obj_idx, (2, WPAD - NEURAL_HI)).reshape(NBLK_NEURAL, 1, TBLK)
    base_p = jnp.pad(base_idx, (0, N_BASE_PAD - base_idx.shape[0]),
                     constant_values=N_ATOMS)

    counts = _hist_kernel(base_p)

    grid_spec = pl.GridSpec(
        grid=(NJ,),
        in_specs=[
            pl.BlockSpec((B, N_OBJ * D), lambda j: (0, 0)),
            pl.BlockSpec((B, TBLK), lambda j: (0, jnp.minimum(j, NBLK_NEURAL - 1))),
            pl.BlockSpec((1, 1, TBLK),
                         lambda j: (jnp.minimum(j, NBLK_NEURAL - 1), 0, 0)),
            pl.BlockSpec((2, TBLK), lambda j: (0, j)),
        ],
        out_specs=pl.BlockSpec((B, TBLK), lambda j: (0, j)),
    )
    return pl.pallas_call(
        _tc_body,
        grid_spec=grid_spec,
        out_shape=jax.ShapeDtypeStruct((B, N_ATOMS), jnp.float32),
    )(zflat, wtp, obj3, counts)


# speedup vs baseline: 34.3789x; 34.3789x over previous
import functools

import jax
import jax.numpy as jnp
from jax import lax
from jax.experimental import pallas as pl
from jax.experimental.pallas import tpu as pltpu
from jax.experimental.pallas import tpu_sc as plsc

B = 32
N_OBJ = 16
D = 32
N_NEURAL = 50000
N_ATOMS = 100000

TBLK = 2048
NJ = (N_ATOMS + TBLK - 1) // TBLK
NEURAL_HI = N_NEURAL + 2
NBLK_NEURAL = (NEURAL_HI + TBLK - 1) // TBLK
WPAD = NBLK_NEURAL * TBLK
NBINS = NJ * TBLK

N_BASE_PAD = 5120
PER_SC = N_BASE_PAD // 2
PER_SUB = PER_SC // 16
CHUNK = PER_SUB // 2
ZSPAN = NBINS // 16


def _hist_body(idx_hbm, out_hbm, idx_a, idx_b, ones_v, buf_v, hist_sh):
    cid = lax.axis_index("c")
    sid = lax.axis_index("s")

    for i in range(CHUNK // 16):
        ones_v[pl.ds(i * 16, 16)] = jnp.full((16,), 1.0, jnp.float32)

    def _zero(i, carry):
        buf_v[pl.ds(i * 16, 16)] = jnp.zeros((16,), jnp.float32)
        return carry

    lax.fori_loop(0, ZSPAN // 16, _zero, 0)

    zbase = sid * ZSPAN
    pltpu.sync_copy(buf_v, hist_sh.at[pl.ds(zbase, ZSPAN)])
    plsc.subcore_barrier()

    base = cid * PER_SC + sid * PER_SUB
    pltpu.sync_copy(idx_hbm.at[pl.ds(base, CHUNK)], idx_a)
    pltpu.sync_copy(idx_hbm.at[pl.ds(base + CHUNK, CHUNK)], idx_b)
    pltpu.sync_copy(ones_v, hist_sh.at[idx_a], add=True)
    pltpu.sync_copy(ones_v, hist_sh.at[idx_b], add=True)
    plsc.subcore_barrier()

    pltpu.sync_copy(hist_sh.at[pl.ds(zbase, ZSPAN)],
                    out_hbm.at[cid, pl.ds(zbase, ZSPAN)])


@functools.cache
def _get_hist_kernel():
    return pl.kernel(
        _hist_body,
        out_type=jax.ShapeDtypeStruct((2, NBINS), jnp.float32),
        mesh=plsc.VectorSubcoreMesh(core_axis_name="c", subcore_axis_name="s"),
        scratch_types=[
            pltpu.VMEM((CHUNK,), jnp.int32),
            pltpu.VMEM((CHUNK,), jnp.int32),
            pltpu.VMEM((CHUNK,), jnp.float32),
            pltpu.VMEM((ZSPAN,), jnp.float32),
            pltpu.VMEM_SHARED((NBINS,), jnp.float32),
        ],
    )


def _tc_body(zf_ref, w_ref, obj_ref, cnt_ref, out_ref):
    j = pl.program_id(0)
    csum = cnt_ref[0:1, :] + cnt_ref[1:2, :]
    col = j * TBLK + lax.broadcasted_iota(jnp.int32, (1, TBLK), 1)

    @pl.when(j < NBLK_NEURAL)
    def _():
        wblk = w_ref[...]
        obj = obj_ref[0]
        wtile = jnp.concatenate([wblk] * N_OBJ, axis=0)
        oid = lax.shift_right_logical(
            lax.broadcasted_iota(jnp.int32, (N_OBJ * D, TBLK), 0), 5)
        u = jnp.where(oid == obj, wtile, 0.0)
        acc = jnp.dot(zf_ref[...], u, preferred_element_type=jnp.float32)
        vals = jax.nn.sigmoid(acc)
        neural = (col >= 2) & (col < NEURAL_HI)
        out = jnp.where(neural, vals, 0.0) + csum
        out_ref[...] = jnp.where(col == 1, 1.0, out)

    @pl.when(j >= NBLK_NEURAL)
    def _():
        out_ref[...] = jnp.broadcast_to(csum, (B, TBLK))


def kernel(Z, atom_idx, obj_idx, w, base_idx):
    del atom_idx
    zflat = Z.reshape(B, N_OBJ * D)
    wtp = jnp.pad(w.astype(jnp.float32).T, ((0, 0), (2, WPAD - NEURAL_HI)))
    obj3 = jnp.pad(obj_idx, (2, WPAD - NEURAL_HI)).reshape(NBLK_NEURAL, 1, TBLK)
    base_p = jnp.pad(base_idx, (0, N_BASE_PAD - base_idx.shape[0]),
                     constant_values=N_ATOMS)

    counts = _get_hist_kernel()(base_p)

    grid_spec = pl.GridSpec(
        grid=(NJ,),
        in_specs=[
            pl.BlockSpec((B, N_OBJ * D), lambda j: (0, 0)),
            pl.BlockSpec((B, TBLK), lambda j: (0, jnp.minimum(j, NBLK_NEURAL - 1))),
            pl.BlockSpec((1, 1, TBLK),
                         lambda j: (jnp.minimum(j, NBLK_NEURAL - 1), 0, 0)),
            pl.BlockSpec((2, TBLK), lambda j: (0, j)),
        ],
        out_specs=pl.BlockSpec((B, TBLK), lambda j: (0, j)),
    )
    return pl.pallas_call(
        _tc_body,
        grid_spec=grid_spec,
        out_shape=jax.ShapeDtypeStruct((B, N_ATOMS), jnp.float32),
    )(zflat, wtp, obj3, counts)


# speedup vs baseline: 35.0297x; 1.0189x over previous
import functools

import jax
import jax.numpy as jnp
from jax import lax
from jax.experimental import pallas as pl
from jax.experimental.pallas import tpu as pltpu
from jax.experimental.pallas import tpu_sc as plsc

B = 32
N_OBJ = 16
D = 32
N_NEURAL = 50000
N_ATOMS = 100000

TBLK = 2048
NJ = (N_ATOMS + TBLK - 1) // TBLK
NEURAL_HI = N_NEURAL + 2
NBLK_NEURAL = (NEURAL_HI + TBLK - 1) // TBLK
WPAD = NBLK_NEURAL * TBLK
NBINS = NJ * TBLK

N_BASE_PAD = 5120
PER_SC = N_BASE_PAD // 2
PER_SUB = PER_SC // 16
CHUNK = PER_SUB // 2
ZSPAN = NBINS // 16


def _hist_body(idx_hbm, out_hbm, idx_a, idx_b, ones_v, buf_v, hist_sh):
    cid = lax.axis_index("c")
    sid = lax.axis_index("s")

    for i in range(CHUNK // 16):
        ones_v[pl.ds(i * 16, 16)] = jnp.full((16,), 1.0, jnp.float32)

    def _zero(i, carry):
        buf_v[pl.ds(i * 16, 16)] = jnp.zeros((16,), jnp.float32)
        return carry

    lax.fori_loop(0, ZSPAN // 16, _zero, 0)

    zbase = sid * ZSPAN
    pltpu.sync_copy(buf_v, hist_sh.at[pl.ds(zbase, ZSPAN)])
    plsc.subcore_barrier()

    base = cid * PER_SC + sid * PER_SUB
    pltpu.sync_copy(idx_hbm.at[pl.ds(base, CHUNK)], idx_a)
    pltpu.sync_copy(idx_hbm.at[pl.ds(base + CHUNK, CHUNK)], idx_b)
    pltpu.sync_copy(ones_v, hist_sh.at[idx_a], add=True)
    pltpu.sync_copy(ones_v, hist_sh.at[idx_b], add=True)
    plsc.subcore_barrier()

    pltpu.sync_copy(hist_sh.at[pl.ds(zbase, ZSPAN)],
                    out_hbm.at[cid, pl.ds(zbase, ZSPAN)])


@functools.cache
def _get_hist_kernel():
    return pl.kernel(
        _hist_body,
        out_type=jax.ShapeDtypeStruct((2, NBINS), jnp.float32),
        mesh=plsc.VectorSubcoreMesh(core_axis_name="c", subcore_axis_name="s"),
        scratch_types=[
            pltpu.VMEM((CHUNK,), jnp.int32),
            pltpu.VMEM((CHUNK,), jnp.int32),
            pltpu.VMEM((CHUNK,), jnp.float32),
            pltpu.VMEM((ZSPAN,), jnp.float32),
            pltpu.VMEM_SHARED((NBINS,), jnp.float32),
        ],
    )


def _tc_body(zf_ref, w_ref, obj_ref, cnt_ref, out_ref):
    j = pl.program_id(0)
    csum = cnt_ref[0:1, :] + cnt_ref[1:2, :]
    col = j * TBLK + lax.broadcasted_iota(jnp.int32, (1, TBLK), 1)

    @pl.when(j < NBLK_NEURAL)
    def _():
        wtb = w_ref[...]
        objr = obj_ref[...]
        parts = [jnp.where(objr == jnp.bfloat16(o), wtb, jnp.bfloat16(0.0))
                 for o in range(N_OBJ)]
        u = jnp.concatenate(parts, axis=0)
        acc = jnp.dot(zf_ref[...], u, preferred_element_type=jnp.float32)
        vals = jax.nn.sigmoid(acc)
        neural = (col >= 2) & (col < NEURAL_HI)
        out = jnp.where(neural, vals, 0.0) + csum
        out_ref[...] = jnp.where(col == 1, 1.0, out)

    @pl.when(j >= NBLK_NEURAL)
    def _():
        out_ref[...] = jnp.broadcast_to(csum, (B, TBLK))


def kernel(Z, atom_idx, obj_idx, w, base_idx):
    del atom_idx
    zflat = Z.reshape(B, N_OBJ * D).astype(jnp.bfloat16)
    wp = jnp.pad(w.astype(jnp.bfloat16).T, ((0, 0), (2, WPAD - NEURAL_HI)))
    obj2 = jnp.pad(obj_idx, (2, WPAD - NEURAL_HI)).astype(jnp.bfloat16)
    obj2 = obj2.reshape(1, WPAD)
    base_p = jnp.pad(base_idx, (0, N_BASE_PAD - base_idx.shape[0]),
                     constant_values=N_ATOMS)

    counts = _get_hist_kernel()(base_p)

    grid_spec = pl.GridSpec(
        grid=(NJ,),
        in_specs=[
            pl.BlockSpec((B, N_OBJ * D), lambda j: (0, 0)),
            pl.BlockSpec((B, TBLK), lambda j: (0, jnp.minimum(j, NBLK_NEURAL - 1))),
            pl.BlockSpec((1, TBLK), lambda j: (0, jnp.minimum(j, NBLK_NEURAL - 1))),
            pl.BlockSpec((2, TBLK), lambda j: (0, j)),
        ],
        out_specs=pl.BlockSpec((B, TBLK), lambda j: (0, j)),
    )
    return pl.pallas_call(
        _tc_body,
        grid_spec=grid_spec,
        out_shape=jax.ShapeDtypeStruct((B, N_ATOMS), jnp.float32),
    )(zflat, wp, obj2, counts)
